# Initial kernel scaffold; baseline (speedup 1.0000x reference)
import jax
import jax.numpy as jnp
from jax.experimental import pallas as pl


def kernel(x, edge_index, W1, b1, W2, b2, Wmu, bmu, Wls, bls):
    raise NotImplementedError("write your pallas kernel here")


# speedup vs baseline: 5.8430x; 5.8430x over previous
import functools

import jax
import jax.numpy as jnp
from jax import lax
from jax.experimental import pallas as pl
from jax.experimental.pallas import tpu as pltpu
from jax.experimental.pallas import tpu_sc as plsc

N = 10000
E = 320000
K = 3

NC = 2
NS = 16
LANES = 16

CHUNK = 128
ACC_ROWS = N + 16
ZROWS = 312
WRITE_ROWS = 624

EPC_F = 20096
E_PAD_F = EPC_F * NS
NCH_F = EPC_F // CHUNK

EPC_E = 10112
E_PAD_E = EPC_E * NC * NS
NCH_E = EPC_E // CHUNK


def _mesh():
  return plsc.VectorSubcoreMesh(core_axis_name="c", subcore_axis_name="s")


def _zero_acc(zero_hbm, acc, s):
  base = s * WRITE_ROWS
  pltpu.sync_copy(zero_hbm, acc.at[pl.ds(base, ZROWS), :])
  pltpu.sync_copy(zero_hbm, acc.at[pl.ds(base + ZROWS, ZROWS), :])

  @pl.when(s == NS - 1)
  def _():
    pltpu.sync_copy(zero_hbm.at[pl.ds(0, 32), :],
                    acc.at[pl.ds(NS * WRITE_ROWS, 32), :])


def _write_out(acc, out_ref, s):
  n0 = s * WRITE_ROWS
  pltpu.sync_copy(acc.at[pl.ds(n0, WRITE_ROWS), :],
                  out_ref.at[pl.ds(n0, WRITE_ROWS), :])

  @pl.when(s == NS - 1)
  def _():
    tail = NS * WRITE_ROWS
    pltpu.sync_copy(acc.at[pl.ds(tail, N - tail), :],
                    out_ref.at[pl.ds(tail, N - tail), :])


def _propagate_fsplit(u2, srcs, dsts):
  fh = 128
  zeros_hbm = jnp.zeros((ZROWS, fh), jnp.float32)

  @functools.partial(
      pl.kernel,
      out_type=jax.ShapeDtypeStruct((2 * N, fh), jnp.float32),
      mesh=_mesh(),
      scratch_types=[
          pltpu.VMEM((CHUNK,), jnp.int32),
          pltpu.VMEM((CHUNK,), jnp.int32),
          pltpu.VMEM((CHUNK,), jnp.int32),
          pltpu.VMEM((CHUNK, fh), jnp.float32),
          pltpu.VMEM_SHARED((ACC_ROWS, fh), jnp.float32),
          pltpu.SemaphoreType.DMA,
      ],
  )
  def prop_kernel(u_hbm, src_hbm, dst_hbm, zero_hbm, out_hbm,
                  sidx, didx, gidx, gbuf, acc, sem):
    c = lax.axis_index("c")
    s = lax.axis_index("s")
    _zero_acc(zero_hbm, acc, s)
    plsc.subcore_barrier()

    row_base = c * N

    def chunk_body(ch, _):
      e0 = s * EPC_F + ch * CHUNK
      pltpu.sync_copy(src_hbm.at[pl.ds(e0, CHUNK)], sidx)
      pltpu.sync_copy(dst_hbm.at[pl.ds(e0, CHUNK)], didx)
      for j in range(CHUNK // LANES):
        sl = pl.ds(j * LANES, LANES)
        gidx[sl] = sidx[sl] + row_base
      pltpu.async_copy(u_hbm.at[gidx], gbuf, sem).wait()
      pltpu.sync_copy(gbuf, acc.at[didx], add=True)
      return 0

    lax.fori_loop(0, NCH_F, chunk_body, 0)
    plsc.subcore_barrier()
    _write_out(acc, out_hbm.at[pl.ds(row_base, N), :], s)

  return prop_kernel(u2, srcs, dsts, zeros_hbm)


def _propagate_esplit(u, srcs, dsts):
  fh = 128
  zeros_hbm = jnp.zeros((ZROWS, fh), jnp.float32)

  @functools.partial(
      pl.kernel,
      out_type=jax.ShapeDtypeStruct((NC, N, fh), jnp.float32),
      mesh=_mesh(),
      scratch_types=[
          pltpu.VMEM((CHUNK,), jnp.int32),
          pltpu.VMEM((CHUNK,), jnp.int32),
          pltpu.VMEM((CHUNK, fh), jnp.float32),
          pltpu.VMEM_SHARED((ACC_ROWS, fh), jnp.float32),
          pltpu.SemaphoreType.DMA,
      ],
  )
  def prop_kernel(u_hbm, src_hbm, dst_hbm, zero_hbm, out_hbm,
                  sidx, didx, gbuf, acc, sem):
    c = lax.axis_index("c")
    s = lax.axis_index("s")
    _zero_acc(zero_hbm, acc, s)
    plsc.subcore_barrier()

    def chunk_body(ch, _):
      e0 = (c * NS + s) * EPC_E + ch * CHUNK
      pltpu.sync_copy(src_hbm.at[pl.ds(e0, CHUNK)], sidx)
      pltpu.sync_copy(dst_hbm.at[pl.ds(e0, CHUNK)], didx)
      pltpu.async_copy(u_hbm.at[sidx], gbuf, sem).wait()
      pltpu.sync_copy(gbuf, acc.at[didx], add=True)
      return 0

    lax.fori_loop(0, NCH_E, chunk_body, 0)
    plsc.subcore_barrier()
    _write_out(acc, out_hbm.at[c], s)

  return prop_kernel(u, srcs, dsts, zeros_hbm)


def _degrees(dsts):
  fh = 128
  zeros_hbm = jnp.zeros((ZROWS, fh), jnp.float32)
  ones_hbm = jnp.ones((CHUNK, fh), jnp.float32)

  @functools.partial(
      pl.kernel,
      out_type=jax.ShapeDtypeStruct((NC, N, fh), jnp.float32),
      mesh=_mesh(),
      scratch_types=[
          pltpu.VMEM((CHUNK,), jnp.int32),
          pltpu.VMEM((CHUNK, fh), jnp.float32),
          pltpu.VMEM_SHARED((ACC_ROWS, fh), jnp.float32),
      ],
  )
  def deg_kernel(dst_hbm, zero_hbm, one_hbm, out_hbm, didx, gbuf, acc):
    c = lax.axis_index("c")
    s = lax.axis_index("s")
    _zero_acc(zero_hbm, acc, s)
    pltpu.sync_copy(one_hbm, gbuf)
    plsc.subcore_barrier()

    def chunk_body(ch, _):
      e0 = (c * NS + s) * EPC_E + ch * CHUNK
      pltpu.sync_copy(dst_hbm.at[pl.ds(e0, CHUNK)], didx)
      pltpu.sync_copy(gbuf, acc.at[didx], add=True)
      return 0

    lax.fori_loop(0, NCH_E, chunk_body, 0)
    plsc.subcore_barrier()
    _write_out(acc, out_hbm.at[c], s)

  return deg_kernel(dsts, zeros_hbm, ones_hbm)


def _matmul(a, w, b, relu):
  m, kd = a.shape
  fout = w.shape[1]
  bm = 1024

  def body(a_ref, w_ref, b_ref, o_ref):
    acc = jnp.dot(a_ref[...], w_ref[...],
                  preferred_element_type=jnp.float32) + b_ref[...]
    if relu:
      acc = jnp.maximum(acc, 0.0)
    o_ref[...] = acc

  return pl.pallas_call(
      body,
      grid=(pl.cdiv(m, bm),),
      in_specs=[
          pl.BlockSpec((bm, kd), lambda i: (i, 0)),
          pl.BlockSpec((kd, fout), lambda i: (0, 0)),
          pl.BlockSpec((1, fout), lambda i: (0, 0)),
      ],
      out_specs=pl.BlockSpec((bm, fout), lambda i: (i, 0)),
      out_shape=jax.ShapeDtypeStruct((m, fout), jnp.float32),
  )(a, w, b.reshape(1, fout))


def _to_split(t):
  n, f = t.shape
  return t.reshape(n, 2, f // 2).transpose(1, 0, 2).reshape(2 * n, f // 2)


def _from_split(t2):
  n2, fh = t2.shape
  n = n2 // 2
  return t2.reshape(2, n, fh).transpose(1, 0, 2).reshape(n, 2 * fh)


def _pad_edges(src, dst, e_pad):
  pad = e_pad - E
  srcs = jnp.concatenate([src, jnp.zeros((pad,), jnp.int32)])
  dsts = jnp.concatenate([dst, jnp.full((pad,), N, jnp.int32)])
  return srcs, dsts


def kernel(x, edge_index, W1, b1, W2, b2, Wmu, bmu, Wls, bls):
  src = edge_index[0].astype(jnp.int32)
  dst = edge_index[1].astype(jnp.int32)
  srcs_f, dsts_f = _pad_edges(src, dst, E_PAD_F)
  srcs_e, dsts_e = _pad_edges(src, dst, E_PAD_E)

  deg = _degrees(dsts_e).sum(axis=0)[:, 0]
  dinv = jnp.where(deg > 0, lax.rsqrt(jnp.maximum(deg, 1e-12)), 0.0)
  dcol = dinv[:, None]
  dsc = dinv[None, :, None]

  def S128(t):
    p = _propagate_esplit(dcol * t, srcs_e, dsts_e).sum(axis=0)
    return dcol * p

  def S256(t2):
    u = (t2.reshape(2, N, 128) * dsc).reshape(2 * N, 128)
    p = _propagate_fsplit(u, srcs_f, dsts_f)
    return (p.reshape(2, N, 128) * dsc).reshape(2 * N, 128)

  h1 = S128(x)
  h2 = S128(h1)
  h3 = S128(h2)
  a1 = jnp.concatenate([x, h1, h2, h3], axis=1)
  w1cat = W1.reshape((K + 1) * 128, 512)
  g1 = _matmul(a1, w1cat, b1, relu=True)

  w2cat = jnp.concatenate([W2[k] for k in range(K + 1)], axis=1)
  y = _matmul(g1, w2cat, jnp.zeros((1024,), jnp.float32), relu=False)
  y0, y1, y2_, y3 = (y[:, 256 * k:256 * (k + 1)] for k in range(4))
  t = S256(_to_split(y3))
  t = S256(_to_split(y2_) + t)
  t = S256(_to_split(y1) + t)
  g2 = jax.nn.relu(y0 + _from_split(t) + b2)

  whcat = jnp.concatenate(
      [jnp.concatenate([Wmu[k], Wls[k]], axis=1) for k in range(K + 1)],
      axis=1)
  z = _matmul(g2, whcat, jnp.zeros((512,), jnp.float32), relu=False)
  z0, z1, z2, z3 = (z[:, 128 * k:128 * (k + 1)] for k in range(4))
  t = S128(z3)
  t = S128(z2 + t)
  t = S128(z1 + t)
  out = z0 + t
  mu = out[:, :64] + bmu
  logstd = out[:, 64:] + bls

  std = jnp.exp(logstd)
  eps = jax.random.normal(jax.random.key(42), std.shape, dtype=std.dtype)
  zlat = eps * std + mu
  return (mu, logstd, zlat)
